# baseline (speedup 1.0000x reference)
import functools

import jax
import jax.numpy as jnp
from jax import lax
from jax.experimental import pallas as pl
from jax.experimental.pallas import tpu as pltpu
from jax.experimental.pallas import tpu_sc as plsc

B = 4
S = 2048
D = 768
N = B * S
NC, NS = 2, 16
NW = NC * NS
WPB = NW // B
TPW = N // NW
CH = 64
NCH = TPW // CH
NJ = D // 16

_mesh = plsc.VectorSubcoreMesh(core_axis_name="c", subcore_axis_name="s")


@functools.partial(
    pl.kernel,
    mesh=_mesh,
    out_type=[
        jax.ShapeDtypeStruct((B, WPB * 16), jnp.float32),
        jax.ShapeDtypeStruct((B, WPB * 16), jnp.float32),
        jax.ShapeDtypeStruct((NW, 16), jnp.float32),
    ],
    scratch_types=[
        pltpu.VMEM((TPW,), jnp.int32),
        pltpu.VMEM((TPW,), jnp.int32),
        pltpu.VMEM((CH, D), jnp.float32),
        pltpu.VMEM((CH, D), jnp.float32),
        pltpu.VMEM((D,), jnp.float32),
        pltpu.VMEM((1, D), jnp.float32),
        pltpu.VMEM((16,), jnp.float32),
        pltpu.VMEM((16,), jnp.float32),
        pltpu.VMEM((16,), jnp.float32),
        pltpu.SemaphoreType.DMA,
        pltpu.SemaphoreType.DMA,
        pltpu.SemaphoreType.DMA,
        pltpu.SemaphoreType.DMA,
    ],
)
def _sc_pool(x_hbm, mask_hbm, table_hbm, w_hbm, p_hbm, d_hbm, e_hbm,
             idx_v, mask_v, rows0, rows1, w_v, e0_v, pout, dout, eout,
             gsem0, gsem1, ssem0, ssem1):
    wid = lax.axis_index("s") * NC + lax.axis_index("c")
    base = wid * TPW

    cp_i = pltpu.async_copy(x_hbm.at[pl.ds(base, TPW)], idx_v, ssem0)
    cp_m = pltpu.async_copy(mask_hbm.at[pl.ds(base, TPW)], mask_v, ssem1)
    cp_i.wait()
    cp_m.wait()

    msum = jnp.zeros((16,), jnp.int32)
    for t in range(TPW // 16):
        sl = pl.ds(t * 16, 16)
        m = mask_v[sl]
        msum = msum + m
        idx_v[sl] = idx_v[sl] * m

    rows = (rows0, rows1)
    gsems = (gsem0, gsem1)
    copies = [None, None]
    copies[0] = pltpu.async_copy(
        table_hbm.at[idx_v.at[pl.ds(0, CH)]], rows[0], gsems[0])

    cp_w = pltpu.async_copy(w_hbm, w_v, ssem0)
    cp_e = pltpu.async_copy(table_hbm.at[pl.ds(0, 1)], e0_v, ssem1)

    accs = tuple(jnp.zeros((16,), jnp.float32) for _ in range(NJ))
    for g in range(NCH):
        if g + 1 < NCH:
            nb = (g + 1) % 2
            copies[nb] = pltpu.async_copy(
                table_hbm.at[idx_v.at[pl.ds((g + 1) * CH, CH)]],
                rows[nb], gsems[nb])
        copies[g % 2].wait()
        rbuf = rows[g % 2]

        def body(r, acc_t):
            return tuple(
                a + rbuf[r, pl.ds(j * 16, 16)] for j, a in enumerate(acc_t))

        accs = lax.fori_loop(0, CH, body, accs, unroll=4)

    cp_w.wait()
    cp_e.wait()

    dot = jnp.zeros((16,), jnp.float32)
    e0w = jnp.zeros((16,), jnp.float32)
    for j in range(NJ):
        wj = w_v[pl.ds(j * 16, 16)]
        dot = dot + accs[j] * wj
        e0w = e0w + e0_v[0, pl.ds(j * 16, 16)] * wj

    pout[...] = dot
    dout[...] = msum.astype(jnp.float32)
    eout[...] = e0w
    brow = wid // WPB
    bcol = (wid % WPB) * 16
    pltpu.sync_copy(pout, p_hbm.at[brow, pl.ds(bcol, 16)])
    pltpu.sync_copy(dout, d_hbm.at[brow, pl.ds(bcol, 16)])
    pltpu.sync_copy(eout, e_hbm.at[wid])


def _finish_body(p_ref, d_ref, e_ref, o_ref):
    num = jnp.sum(p_ref[...], axis=1)
    cnt = jnp.sum(d_ref[...], axis=1)
    e0w = jnp.sum(e_ref[...][0:1, 0:16])
    num = num - (S - cnt) * e0w
    o_ref[...] = num / jnp.clip(cnt, 1e-5, None)


def kernel(x, mask, embedding_table, prompt_embed, response_embed, W_pred):
    x_f = x.reshape(N)
    if x_f.dtype != jnp.int32:
        x_f = x_f.astype(jnp.int32)
    mask_f = mask.astype(jnp.int32).reshape(N)
    p, d, e = _sc_pool(x_f, mask_f, embedding_table, W_pred)
    pred = pl.pallas_call(
        _finish_body,
        out_shape=jax.ShapeDtypeStruct((B,), jnp.float32),
    )(p, d, e)
    return pred


# speedup vs baseline: 1.3314x; 1.3314x over previous
import functools

import jax
import jax.numpy as jnp
from jax import lax
from jax.experimental import pallas as pl
from jax.experimental.pallas import tpu as pltpu
from jax.experimental.pallas import tpu_sc as plsc

B = 4
S = 2048
D = 768
N = B * S
NC, NS = 2, 16
NW = NC * NS
WPB = NW // B
TPW = N // NW
CH = 64
NCH = TPW // CH
NJ = D // 16

_mesh = plsc.VectorSubcoreMesh(core_axis_name="c", subcore_axis_name="s")


@functools.partial(
    pl.kernel,
    mesh=_mesh,
    out_type=[
        jax.ShapeDtypeStruct((B, WPB * 16), jnp.float32),
        jax.ShapeDtypeStruct((B, WPB * 16), jnp.float32),
        jax.ShapeDtypeStruct((NW, 16), jnp.float32),
    ],
    scratch_types=[
        pltpu.VMEM((NCH, CH), jnp.int32),
        pltpu.VMEM((TPW,), jnp.int32),
        pltpu.VMEM((CH, D), jnp.float32),
        pltpu.VMEM((CH, D), jnp.float32),
        pltpu.VMEM((D,), jnp.float32),
        pltpu.VMEM((1, D), jnp.float32),
        pltpu.VMEM((16,), jnp.float32),
        pltpu.VMEM((16,), jnp.float32),
        pltpu.VMEM((16,), jnp.float32),
        pltpu.SemaphoreType.DMA,
        pltpu.SemaphoreType.DMA,
        pltpu.SemaphoreType.DMA,
        pltpu.SemaphoreType.DMA,
    ],
)
def _sc_pool(x_hbm, mask_hbm, table_hbm, w_hbm, p_hbm, d_hbm, e_hbm,
             idx_v, mask_v, rows0, rows1, w_v, e0_v, pout, dout, eout,
             gsem0, gsem1, ssem0, ssem1):
    wid = lax.axis_index("s") * NC + lax.axis_index("c")
    base = wid * TPW

    cp_is = [
        pltpu.async_copy(x_hbm.at[pl.ds(base + g * CH, CH)], idx_v.at[g],
                         ssem0)
        for g in range(NCH)
    ]
    cp_m = pltpu.async_copy(mask_hbm.at[pl.ds(base, TPW)], mask_v, ssem1)
    for cp in cp_is:
        cp.wait()
    cp_m.wait()

    msum = jnp.zeros((16,), jnp.int32)
    for g in range(NCH):
        for t in range(CH // 16):
            m = mask_v[pl.ds(g * CH + t * 16, 16)]
            msum = msum + m
            sl = pl.ds(t * 16, 16)
            idx_v[g, sl] = idx_v[g, sl] * m

    rows = (rows0, rows1)
    gsems = (gsem0, gsem1)
    copies = [None, None]
    copies[0] = pltpu.async_copy(table_hbm.at[idx_v.at[0]], rows[0], gsems[0])

    cp_w = pltpu.async_copy(w_hbm, w_v, ssem0)
    cp_e = pltpu.async_copy(table_hbm.at[pl.ds(0, 1)], e0_v, ssem1)

    accs = tuple(jnp.zeros((16,), jnp.float32) for _ in range(NJ))
    for g in range(NCH):
        if g + 1 < NCH:
            nb = (g + 1) % 2
            copies[nb] = pltpu.async_copy(
                table_hbm.at[idx_v.at[g + 1]], rows[nb], gsems[nb])
        copies[g % 2].wait()
        rbuf = rows[g % 2]

        def body(r, acc_t):
            return tuple(
                a + rbuf[r, pl.ds(j * 16, 16)] for j, a in enumerate(acc_t))

        accs = lax.fori_loop(0, CH, body, accs)

    cp_w.wait()
    cp_e.wait()

    dot = jnp.zeros((16,), jnp.float32)
    e0w = jnp.zeros((16,), jnp.float32)
    for j in range(NJ):
        wj = w_v[pl.ds(j * 16, 16)]
        dot = dot + accs[j] * wj
        e0w = e0w + e0_v[0, pl.ds(j * 16, 16)] * wj

    pout[...] = dot
    dout[...] = msum.astype(jnp.float32)
    eout[...] = e0w
    brow = wid // WPB
    bcol = (wid % WPB) * 16
    pltpu.sync_copy(pout, p_hbm.at[brow, pl.ds(bcol, 16)])
    pltpu.sync_copy(dout, d_hbm.at[brow, pl.ds(bcol, 16)])
    pltpu.sync_copy(eout, e_hbm.at[wid])


def _finish_body(p_ref, d_ref, e_ref, o_ref):
    num = jnp.sum(p_ref[...], axis=1)
    cnt = jnp.sum(d_ref[...], axis=1)
    e0w = jnp.sum(e_ref[...][0:1, 0:16])
    num = num - (S - cnt) * e0w
    o_ref[...] = num / jnp.clip(cnt, 1e-5, None)


def kernel(x, mask, embedding_table, prompt_embed, response_embed, W_pred):
    x_f = x.reshape(N)
    if x_f.dtype != jnp.int32:
        x_f = x_f.astype(jnp.int32)
    mask_f = mask.astype(jnp.int32).reshape(N)
    p, d, e = _sc_pool(x_f, mask_f, embedding_table, W_pred)
    pred = pl.pallas_call(
        _finish_body,
        out_shape=jax.ShapeDtypeStruct((B,), jnp.float32),
    )(p, d, e)
    return pred


# speedup vs baseline: 1.3392x; 1.0059x over previous
import functools

import jax
import jax.numpy as jnp
from jax import lax
from jax.experimental import pallas as pl
from jax.experimental.pallas import tpu as pltpu
from jax.experimental.pallas import tpu_sc as plsc

B = 4
S = 2048
D = 768
N = B * S
NC, NS = 2, 16
NW = NC * NS
WPB = NW // B
TPW = N // NW
CH = 64
NCH = TPW // CH
NJ = D // 16

_mesh = plsc.VectorSubcoreMesh(core_axis_name="c", subcore_axis_name="s")


@functools.partial(
    pl.kernel,
    mesh=_mesh,
    out_type=[
        jax.ShapeDtypeStruct((B, WPB * 16), jnp.float32),
        jax.ShapeDtypeStruct((NW, 16), jnp.float32),
    ],
    scratch_types=[
        pltpu.VMEM((NCH, CH), jnp.int32),
        pltpu.VMEM((CH, D), jnp.float32),
        pltpu.VMEM((CH, D), jnp.float32),
        pltpu.VMEM((D,), jnp.float32),
        pltpu.VMEM((1, D), jnp.float32),
        pltpu.VMEM((16,), jnp.float32),
        pltpu.VMEM((16,), jnp.float32),
        pltpu.SemaphoreType.DMA,
        pltpu.SemaphoreType.DMA,
        pltpu.SemaphoreType.DMA,
        pltpu.SemaphoreType.DMA,
    ],
)
def _sc_pool(x_hbm, table_hbm, w_hbm, p_hbm, e_hbm,
             idx_v, rows0, rows1, w_v, e0_v, pout, eout,
             gsem0, gsem1, ssem0, ssem1):
    wid = lax.axis_index("s") * NC + lax.axis_index("c")
    brow = wid // WPB
    scol = (wid % WPB) * TPW

    cp_is = [
        pltpu.async_copy(x_hbm.at[brow, pl.ds(scol + g * CH, CH)],
                         idx_v.at[g], ssem0)
        for g in range(NCH)
    ]
    cp_is[0].wait()

    rows = (rows0, rows1)
    gsems = (gsem0, gsem1)
    copies = [None, None]
    copies[0] = pltpu.async_copy(table_hbm.at[idx_v.at[0]], rows[0], gsems[0])

    cp_w = pltpu.async_copy(w_hbm, w_v, ssem1)
    cp_e = pltpu.async_copy(table_hbm.at[pl.ds(0, 1)], e0_v, ssem1)
    for cp in cp_is[1:]:
        cp.wait()

    accs = tuple(jnp.zeros((16,), jnp.float32) for _ in range(NJ))
    for g in range(NCH):
        if g + 1 < NCH:
            nb = (g + 1) % 2
            copies[nb] = pltpu.async_copy(
                table_hbm.at[idx_v.at[g + 1]], rows[nb], gsems[nb])
        copies[g % 2].wait()
        rbuf = rows[g % 2]

        def body(r, acc_t):
            return tuple(
                a + rbuf[r, pl.ds(j * 16, 16)] for j, a in enumerate(acc_t))

        accs = lax.fori_loop(0, CH, body, accs)

    cp_w.wait()
    cp_e.wait()

    dot = jnp.zeros((16,), jnp.float32)
    e0w = jnp.zeros((16,), jnp.float32)
    for j in range(NJ):
        wj = w_v[pl.ds(j * 16, 16)]
        dot = dot + accs[j] * wj
        e0w = e0w + e0_v[0, pl.ds(j * 16, 16)] * wj

    pout[...] = dot
    eout[...] = e0w
    pltpu.sync_copy(pout, p_hbm.at[brow, pl.ds((wid % WPB) * 16, 16)])
    pltpu.sync_copy(eout, e_hbm.at[wid])


def _finish_body(p_ref, e_ref, m_ref, o_ref):
    num = jnp.sum(p_ref[...], axis=1)
    cnt = jnp.sum(m_ref[...].astype(jnp.float32), axis=1)
    e0w = jnp.sum(e_ref[...][0:1, 0:16])
    num = num - (S - cnt) * e0w
    o_ref[...] = num / jnp.clip(cnt, 1e-5, None)


def kernel(x, mask, embedding_table, prompt_embed, response_embed, W_pred):
    xm = jnp.where(mask, x.astype(jnp.int32), 0)
    p, e = _sc_pool(xm, embedding_table, W_pred)
    pred = pl.pallas_call(
        _finish_body,
        out_shape=jax.ShapeDtypeStruct((B,), jnp.float32),
    )(p, e, mask)
    return pred


# speedup vs baseline: 1.3485x; 1.0070x over previous
import functools

import jax
import jax.numpy as jnp
from jax import lax
from jax.experimental import pallas as pl
from jax.experimental.pallas import tpu as pltpu
from jax.experimental.pallas import tpu_sc as plsc

B = 4
S = 2048
D = 768
N = B * S
NC, NS = 2, 16
NW = NC * NS
WPB = NW // B
TPW = N // NW
CH = 64
NCH = TPW // CH
NJ = D // 16

_mesh = plsc.VectorSubcoreMesh(core_axis_name="c", subcore_axis_name="s")


@functools.partial(
    pl.kernel,
    mesh=_mesh,
    out_type=jax.ShapeDtypeStruct((B, WPB * 16), jnp.float32),
    scratch_types=[
        pltpu.VMEM((NCH, CH), jnp.int32),
        pltpu.VMEM((CH, D), jnp.float32),
        pltpu.VMEM((CH, D), jnp.float32),
        pltpu.VMEM((D,), jnp.float32),
        pltpu.VMEM((16,), jnp.float32),
        pltpu.SemaphoreType.DMA,
        pltpu.SemaphoreType.DMA,
        pltpu.SemaphoreType.DMA,
        pltpu.SemaphoreType.DMA,
    ],
)
def _sc_pool(x_hbm, table_hbm, w_hbm, p_hbm,
             idx_v, rows0, rows1, w_v, pout,
             gsem0, gsem1, ssem0, ssem1):
    wid = lax.axis_index("s") * NC + lax.axis_index("c")
    brow = wid // WPB
    scol = (wid % WPB) * TPW

    cp_is = [
        pltpu.async_copy(x_hbm.at[brow, pl.ds(scol + g * CH, CH)],
                         idx_v.at[g], ssem0)
        for g in range(NCH)
    ]
    cp_is[0].wait()

    rows = (rows0, rows1)
    gsems = (gsem0, gsem1)
    copies = [None, None]
    copies[0] = pltpu.async_copy(table_hbm.at[idx_v.at[0]], rows[0], gsems[0])

    cp_w = pltpu.async_copy(w_hbm, w_v, ssem1)
    for cp in cp_is[1:]:
        cp.wait()

    accs = tuple(jnp.zeros((16,), jnp.float32) for _ in range(NJ))
    for g in range(NCH):
        if g + 1 < NCH:
            nb = (g + 1) % 2
            copies[nb] = pltpu.async_copy(
                table_hbm.at[idx_v.at[g + 1]], rows[nb], gsems[nb])
        copies[g % 2].wait()
        rbuf = rows[g % 2]

        def body(r, acc_t):
            return tuple(
                a + rbuf[r, pl.ds(j * 16, 16)] for j, a in enumerate(acc_t))

        accs = lax.fori_loop(0, CH, body, accs)

    cp_w.wait()

    dot = jnp.zeros((16,), jnp.float32)
    for j in range(NJ):
        dot = dot + accs[j] * w_v[pl.ds(j * 16, 16)]

    pout[...] = dot
    pltpu.sync_copy(pout, p_hbm.at[brow, pl.ds((wid % WPB) * 16, 16)])


def _finish_body(p_ref, m_ref, e0_ref, w_ref, o_ref):
    num = jnp.sum(p_ref[...], axis=1)
    cnt = jnp.sum(m_ref[...].astype(jnp.float32), axis=1)
    e0w = jnp.sum(e0_ref[...] * w_ref[...][None, :])
    num = num - (S - cnt) * e0w
    o_ref[...] = num / jnp.clip(cnt, 1e-5, None)


def kernel(x, mask, embedding_table, prompt_embed, response_embed, W_pred):
    xm = jnp.where(mask, x.astype(jnp.int32), 0)
    p = _sc_pool(xm, embedding_table, W_pred)
    pred = pl.pallas_call(
        _finish_body,
        out_shape=jax.ShapeDtypeStruct((B,), jnp.float32),
    )(p, mask, embedding_table[0:1], W_pred)
    return pred
